# baseline (speedup 1.0000x reference)
import functools
import math

import jax
import jax.numpy as jnp
from jax import lax
from jax.experimental import pallas as pl
from jax.experimental.pallas import tpu as pltpu
from jax.experimental.pallas import tpu_sc as plsc

N = 10000
E = 320000
HID = 128
K = 512
HEADS = 4
DH = HID // HEADS
SCALE = 1.0 / math.sqrt(DH)

NC = 2
NS = 16
CH = 128
NCH = 80
PW = CH * NCH
EPAD = NC * NS * PW
NPAD = 10240
RPT = NPAD // NS

RB = 1000
GRID = N // RB


NB = 2


def _sc_deg_body(dst2, zr, on, deg_out, dall, onesb, dacc):
    cid = lax.axis_index("c")
    sid = lax.axis_index("s")
    row0 = sid * RPT
    pltpu.sync_copy(zr, dacc.at[pl.ds(row0, RPT)])
    pltpu.sync_copy(on, onesb)
    gbase = (cid * NS + sid) * NCH
    pltpu.sync_copy(dst2.at[pl.ds(gbase, NCH)], dall)
    plsc.subcore_barrier()

    def step(c, carry):
        pltpu.sync_copy(onesb, dacc.at[dall.at[c]], add=True)
        return carry

    lax.fori_loop(0, NCH, step, 0)
    plsc.subcore_barrier()
    obase = cid * NPAD + row0
    pltpu.sync_copy(dacc.at[pl.ds(row0, RPT)], deg_out.at[pl.ds(obase, RPT)])


SCH = 16
HEAVY = 160
LIGHT = 2 * NCH - HEAVY


def _sc_edge_body(hn, src2, dst2, zr, sum_out, sall, dall, rows0, rows1,
                  acc, sem0, sem1):
    cid = lax.axis_index("c")
    sid = lax.axis_index("s")
    row0 = sid * RPT
    pltpu.sync_copy(zr, acc.at[pl.ds(row0, RPT)])
    n_my = jnp.where(cid == 0, HEAVY, LIGHT)
    gbase = jnp.where(cid == 0, sid * HEAVY, NS * HEAVY + sid * LIGHT)
    plsc.subcore_barrier()
    rows = (rows0, rows1)
    sems = (sem0, sem1)

    def souter(s, carry):
        sb = gbase + s * SCH
        pltpu.sync_copy(src2.at[pl.ds(sb, SCH)], sall)
        pltpu.sync_copy(dst2.at[pl.ds(sb, SCH)], dall)
        for b in range(NB):
            pltpu.async_copy(hn.at[sall.at[b]], rows[b], sems[b])

        def gstep(g, c2):
            c0 = g * NB
            for b in range(NB):
                cc = c0 + b
                pltpu.make_async_copy(hn.at[sall.at[cc]], rows[b],
                                      sems[b]).wait()
                pltpu.sync_copy(rows[b], acc.at[dall.at[cc]], add=True)

                @pl.when(cc + NB < SCH)
                def _issue(cc=cc, b=b):
                    pltpu.async_copy(hn.at[sall.at[cc + NB]], rows[b], sems[b])
            return c2

        lax.fori_loop(0, SCH // NB, gstep, 0)
        return carry

    lax.fori_loop(0, n_my // SCH, souter, 0)
    plsc.subcore_barrier()
    obase = cid * NPAD + row0
    pltpu.sync_copy(acc.at[pl.ds(row0, RPT)], sum_out.at[pl.ds(obase, RPT)])


@functools.lru_cache(maxsize=None)
def _get_sc_kernels():
    mesh = plsc.VectorSubcoreMesh(core_axis_name="c", subcore_axis_name="s")
    sc_deg = pl.kernel(
        _sc_deg_body,
        out_type=jax.ShapeDtypeStruct((NC * NPAD, HID), jnp.float32),
        mesh=mesh,
        scratch_types=[
            pltpu.VMEM((NCH, CH), jnp.int32),
            pltpu.VMEM((CH, HID), jnp.float32),
            pltpu.VMEM_SHARED((NPAD, HID), jnp.float32),
        ],
    )
    sc_edge = pl.kernel(
        _sc_edge_body,
        out_type=jax.ShapeDtypeStruct((NC * NPAD, HID), jnp.float32),
        mesh=mesh,
        scratch_types=[
            pltpu.VMEM((SCH, CH), jnp.int32),
            pltpu.VMEM((SCH, CH), jnp.int32),
            pltpu.VMEM((CH, HID), jnp.float32),
            pltpu.VMEM((CH, HID), jnp.float32),
            pltpu.VMEM_SHARED((NPAD, HID), jnp.float32),
            pltpu.SemaphoreType.DMA,
            pltpu.SemaphoreType.DMA,
        ],
    )
    return sc_deg, sc_edge


def _relu_ln(h, g, b):
    mu = jnp.mean(h, axis=-1, keepdims=True)
    var = jnp.mean((h - mu) ** 2, axis=-1, keepdims=True)
    return jax.nn.relu((h - mu) / jnp.sqrt(var + 1e-5) * g + b)


def _pre_body(x_ref, w1_ref, b1_ref, g_ref, bb_ref, wp_ref, bp_ref,
              wq_ref, bq_ref, hn_ref, q_ref):
    x = x_ref[...]
    h = jnp.dot(x, w1_ref[...], preferred_element_type=jnp.float32) + b1_ref[...]
    hn_ref[...] = _relu_ln(h, g_ref[...], bb_ref[...])
    qx = jnp.dot(x, wp_ref[...], preferred_element_type=jnp.float32) + bp_ref[...]
    q_ref[...] = jnp.dot(qx, wq_ref[...], preferred_element_type=jnp.float32) + bq_ref[...]


def _comm_body(ntc_ref, x_ref, sa_ref):
    i = pl.program_id(0)

    @pl.when(i == 0)
    def _init():
        sa_ref[...] = jnp.zeros_like(sa_ref)

    ntc = ntc_ref[0, 0, :]
    onehot_t = (lax.broadcasted_iota(jnp.int32, (K, RB), 0)
                == ntc[None, :]).astype(jnp.float32)
    xa = jnp.concatenate([x_ref[...], jnp.ones((RB, HID), jnp.float32)],
                         axis=-1)
    sa_ref[...] += jnp.dot(onehot_t, xa, preferred_element_type=jnp.float32)


def _combine_body(s0_ref, s1_ref, d0_ref, d1_ref, hn_ref, wl_ref, bl_ref,
                  wr_ref, g_ref, bb_ref, out_ref):
    deg = jnp.clip(d0_ref[...][:, 0:1] + d1_ref[...][:, 0:1], 1.0, None)
    agg = (s0_ref[...] + s1_ref[...]) / deg
    hn = hn_ref[...]
    h = (jnp.dot(agg, wl_ref[...], preferred_element_type=jnp.float32)
         + bl_ref[...]
         + jnp.dot(hn, wr_ref[...], preferred_element_type=jnp.float32))
    out_ref[...] = _relu_ln(h, g_ref[...], bb_ref[...])


def _attn_body(s0_ref, s1_ref, d0_ref, d1_ref, hn_ref, wl_ref, bl_ref,
               wr_ref, g_ref, bb_ref,
               q_ref, dm_ref, sa_ref, wk_ref, bk_ref, wv_ref, bv_ref,
               w2_ref, b2_ref, wd_ref, bd_ref, out_ref):
    deg = jnp.clip(d0_ref[...][:, 0:1] + d1_ref[...][:, 0:1], 1.0, None)
    agg = (s0_ref[...] + s1_ref[...]) / deg
    hcomb = (jnp.dot(agg, wl_ref[...], preferred_element_type=jnp.float32)
             + bl_ref[...]
             + jnp.dot(hn_ref[...], wr_ref[...],
                       preferred_element_type=jnp.float32))
    hfin = _relu_ln(hcomb, g_ref[...], bb_ref[...])
    sa = sa_ref[...]
    cnt = sa[:, HID:HID + 1]
    avg = sa[:, :HID] / jnp.clip(cnt, 1.0, None)
    km = jnp.dot(avg, wk_ref[...], preferred_element_type=jnp.float32) + bk_ref[...]
    vm = jnp.dot(avg, wv_ref[...], preferred_element_type=jnp.float32) + bv_ref[...]
    lc = jnp.log(cnt)
    dmf = dm_ref[...].astype(jnp.float32) * wd_ref[0, 0] + bd_ref[0, 0]
    w2 = w2_ref[...]
    acc = (jnp.dot(hfin, w2[:HID, :], preferred_element_type=jnp.float32)
           + b2_ref[...])
    qs = q_ref[...] * SCALE
    ones_col = jnp.ones((RB, 1), jnp.float32)
    for h in range(HEADS):
        c0 = h * DH
        qh = jnp.concatenate([qs[:, c0:c0 + DH], ones_col], axis=1)
        kh = jnp.concatenate([km[:, c0:c0 + DH], lc], axis=1)
        dots = lax.dot_general(qh, kh, (((1,), (1,)), ((), ())),
                               preferred_element_type=jnp.float32) + dmf
        m = jnp.max(dots, axis=-1, keepdims=True)
        e = jnp.exp(dots - m)
        p = e / jnp.sum(e, axis=-1, keepdims=True)
        oh = jnp.dot(p, vm[:, c0:c0 + DH], preferred_element_type=jnp.float32)
        acc += jnp.dot(oh, w2[HID + c0:HID + c0 + DH, :],
                       preferred_element_type=jnp.float32)
    out_ref[...] = acc


def _row_spec():
    return pl.BlockSpec((RB, HID), lambda i: (i, 0))


def _const_spec(shape):
    return pl.BlockSpec(shape, lambda i: tuple(0 for _ in shape))


def _pre_call(x, w1, b1, g, b, wp, bp, wq, bq):
    return pl.pallas_call(
        _pre_body,
        grid=(GRID,),
        in_specs=[_row_spec()] + [_const_spec(a.shape)
                                  for a in (w1, b1, g, b, wp, bp, wq, bq)],
        out_specs=[_row_spec(), _row_spec()],
        out_shape=[jax.ShapeDtypeStruct((N, HID), jnp.float32)] * 2,
    )(x, w1, b1, g, b, wp, bp, wq, bq)


def _comm_call(ntc3, x):
    return pl.pallas_call(
        _comm_body,
        grid=(GRID,),
        in_specs=[pl.BlockSpec((1, 1, RB), lambda i: (i, 0, 0)), _row_spec()],
        out_specs=_const_spec((K, 2 * HID)),
        out_shape=jax.ShapeDtypeStruct((K, 2 * HID), jnp.float32),
    )(ntc3, x)


def _combine_call(s0, s1, d0, d1, hn, wl, bl, wr, g, b):
    return pl.pallas_call(
        _combine_body,
        grid=(GRID,),
        in_specs=[_row_spec(), _row_spec(),
                  pl.BlockSpec((RB, 16), lambda i: (i, 0)),
                  pl.BlockSpec((RB, 16), lambda i: (i, 0)),
                  _row_spec()] + [_const_spec(a.shape)
                                  for a in (wl, bl, wr, g, b)],
        out_specs=_row_spec(),
        out_shape=jax.ShapeDtypeStruct((N, HID), jnp.float32),
    )(s0, s1, d0, d1, hn, wl, bl, wr, g, b)


def _attn_call(s0, s1, d0, d1, hn, wl, bl, wr, g, b,
               q, dm, sa, wk, bk, wv, bv, w2, b2, wd, bd):
    return pl.pallas_call(
        _attn_body,
        grid=(GRID,),
        in_specs=[_row_spec(), _row_spec(),
                  pl.BlockSpec((RB, 16), lambda i: (i, 0)),
                  pl.BlockSpec((RB, 16), lambda i: (i, 0)),
                  _row_spec()]
                 + [_const_spec(a.shape) for a in (wl, bl, wr, g, b)]
                 + [_row_spec(), pl.BlockSpec((RB, K), lambda i: (i, 0))]
                 + [_const_spec(a.shape)
                    for a in (sa, wk, bk, wv, bv, w2, b2, wd, bd)],
        out_specs=_row_spec(),
        out_shape=jax.ShapeDtypeStruct((N, HID), jnp.float32),
    )(s0, s1, d0, d1, hn, wl, bl, wr, g, b,
      q, dm, sa, wk, bk, wv, bv, w2, b2, wd, bd)


def kernel(x, edge_index, distance_matrix, nodes_to_community_tensor, W1, b1,
           conv_gamma, conv_beta, Wl, bl, Wr, gamma, beta, Wp, bp, Wq, bq, Wk,
           bk, Wv, bv, w_dis, b_dis, W2, b2):
    r = lambda v: v.reshape(1, -1)
    pad = EPAD - E
    src_p = jnp.concatenate([edge_index[0], jnp.zeros((pad,), jnp.int32)])
    dst_p = jnp.concatenate([edge_index[1], jnp.full((pad,), N, jnp.int32)])
    src_p = src_p.reshape(EPAD // CH, CH)
    dst_p = dst_p.reshape(EPAD // CH, CH)
    zr = jnp.zeros((RPT, HID), jnp.float32)
    on = jnp.ones((CH, HID), jnp.float32)

    sc_deg, sc_edge = _get_sc_kernels()

    ntc3 = nodes_to_community_tensor.reshape(GRID, 1, RB)
    hn1, q = _pre_call(x, W1, r(b1), r(conv_gamma[0]), r(conv_beta[0]),
                       Wp, r(bp), Wq, r(bq))
    sa = _comm_call(ntc3, x)

    degs = sc_deg(dst_p, zr, on)
    d0 = degs[:N, :16]
    d1 = degs[NPAD:NPAD + N, :16]
    sums1 = sc_edge(hn1, src_p, dst_p, zr)
    hn2 = _combine_call(sums1[:N], sums1[NPAD:NPAD + N], d0, d1, hn1,
                        Wl[0], r(bl[0]), Wr[0], r(conv_gamma[1]),
                        r(conv_beta[1]))

    sums2 = sc_edge(hn2, src_p, dst_p, zr)

    out = _attn_call(sums2[:N], sums2[NPAD:NPAD + N], d0, d1, hn2,
                     Wl[1], r(bl[1]), Wr[1], r(gamma), r(beta),
                     q, distance_matrix, sa, Wk, r(bk), Wv, r(bv),
                     W2, r(b2), w_dis.reshape(1, 1), b_dis.reshape(1, 1))
    return out


# speedup vs baseline: 1.3070x; 1.3070x over previous
import functools
import math

import jax
import jax.numpy as jnp
from jax import lax
from jax.experimental import pallas as pl
from jax.experimental.pallas import tpu as pltpu
from jax.experimental.pallas import tpu_sc as plsc

N = 10000
E = 320000
HID = 128
K = 512
HEADS = 4
DH = HID // HEADS
SCALE = 1.0 / math.sqrt(DH)

NC = 2
NS = 16
CH = 128
NCH = 80
PW = CH * NCH
EPAD = NC * NS * PW
NPAD = 10240
RPT = NPAD // NS

RB = 1000
GRID = N // RB


NB = 2


def _sc_deg_body(dst2, zr, on, deg_out, dall, onesb, dacc):
    cid = lax.axis_index("c")
    sid = lax.axis_index("s")
    row0 = sid * RPT
    pltpu.sync_copy(zr, dacc.at[pl.ds(row0, RPT)])
    pltpu.sync_copy(on, onesb)
    gbase = (cid * NS + sid) * NCH
    pltpu.sync_copy(dst2.at[pl.ds(gbase, NCH)], dall)
    plsc.subcore_barrier()

    def step(c, carry):
        pltpu.sync_copy(onesb, dacc.at[dall.at[c]], add=True)
        return carry

    lax.fori_loop(0, NCH, step, 0)
    plsc.subcore_barrier()
    obase = cid * NPAD + row0
    pltpu.sync_copy(dacc.at[pl.ds(row0, RPT)], deg_out.at[pl.ds(obase, RPT)])


SCH = 8
HEAVY = 152
LIGHT = 2 * NCH - HEAVY


def _sc_edge_body(hn, src2, dst2, zr, sum_out, sall, dall, rows0, rows1,
                  acc, sem0, sem1):
    cid = lax.axis_index("c")
    sid = lax.axis_index("s")
    row0 = sid * RPT
    pltpu.sync_copy(zr, acc.at[pl.ds(row0, RPT)])
    n_my = jnp.where(cid == 0, HEAVY, LIGHT)
    gbase = jnp.where(cid == 0, sid * HEAVY, NS * HEAVY + sid * LIGHT)
    plsc.subcore_barrier()
    rows = (rows0, rows1)
    sems = (sem0, sem1)

    def souter(s, carry):
        sb = gbase + s * SCH
        pltpu.sync_copy(src2.at[pl.ds(sb, SCH)], sall)
        pltpu.sync_copy(dst2.at[pl.ds(sb, SCH)], dall)
        for b in range(NB):
            pltpu.async_copy(hn.at[sall.at[b]], rows[b], sems[b])

        def gstep(g, c2):
            c0 = g * NB
            for b in range(NB):
                cc = c0 + b
                pltpu.make_async_copy(hn.at[sall.at[cc]], rows[b],
                                      sems[b]).wait()
                pltpu.sync_copy(rows[b], acc.at[dall.at[cc]], add=True)

                @pl.when(cc + NB < SCH)
                def _issue(cc=cc, b=b):
                    pltpu.async_copy(hn.at[sall.at[cc + NB]], rows[b], sems[b])
            return c2

        lax.fori_loop(0, SCH // NB, gstep, 0)
        return carry

    lax.fori_loop(0, n_my // SCH, souter, 0)
    plsc.subcore_barrier()
    obase = cid * NPAD + row0
    pltpu.sync_copy(acc.at[pl.ds(row0, RPT)], sum_out.at[pl.ds(obase, RPT)])


@functools.lru_cache(maxsize=None)
def _get_sc_kernels():
    mesh = plsc.VectorSubcoreMesh(core_axis_name="c", subcore_axis_name="s")
    sc_deg = pl.kernel(
        _sc_deg_body,
        out_type=jax.ShapeDtypeStruct((NC * NPAD, HID), jnp.float32),
        mesh=mesh,
        scratch_types=[
            pltpu.VMEM((NCH, CH), jnp.int32),
            pltpu.VMEM((CH, HID), jnp.float32),
            pltpu.VMEM_SHARED((NPAD, HID), jnp.float32),
        ],
    )
    sc_edge = pl.kernel(
        _sc_edge_body,
        out_type=jax.ShapeDtypeStruct((NC * NPAD, HID), jnp.float32),
        mesh=mesh,
        scratch_types=[
            pltpu.VMEM((SCH, CH), jnp.int32),
            pltpu.VMEM((SCH, CH), jnp.int32),
            pltpu.VMEM((CH, HID), jnp.float32),
            pltpu.VMEM((CH, HID), jnp.float32),
            pltpu.VMEM_SHARED((NPAD, HID), jnp.float32),
            pltpu.SemaphoreType.DMA,
            pltpu.SemaphoreType.DMA,
        ],
    )
    return sc_deg, sc_edge


def _relu_ln(h, g, b):
    mu = jnp.mean(h, axis=-1, keepdims=True)
    var = jnp.mean((h - mu) ** 2, axis=-1, keepdims=True)
    return jax.nn.relu((h - mu) / jnp.sqrt(var + 1e-5) * g + b)


def _pre_body(x_ref, w1_ref, b1_ref, g_ref, bb_ref, wp_ref, bp_ref,
              wq_ref, bq_ref, hn_ref, q_ref):
    x = x_ref[...]
    h = jnp.dot(x, w1_ref[...], preferred_element_type=jnp.float32) + b1_ref[...]
    hn_ref[...] = _relu_ln(h, g_ref[...], bb_ref[...])
    qx = jnp.dot(x, wp_ref[...], preferred_element_type=jnp.float32) + bp_ref[...]
    q_ref[...] = jnp.dot(qx, wq_ref[...], preferred_element_type=jnp.float32) + bq_ref[...]


def _comm_body(ntc_ref, x_ref, sa_ref):
    i = pl.program_id(0)

    @pl.when(i == 0)
    def _init():
        sa_ref[...] = jnp.zeros_like(sa_ref)

    ntc = ntc_ref[0, 0, :]
    onehot_t = (lax.broadcasted_iota(jnp.int32, (K, RB), 0)
                == ntc[None, :]).astype(jnp.float32)
    xa = jnp.concatenate([x_ref[...], jnp.ones((RB, HID), jnp.float32)],
                         axis=-1)
    sa_ref[...] += jnp.dot(onehot_t, xa, preferred_element_type=jnp.float32)


def _combine_body(s0_ref, s1_ref, d0_ref, d1_ref, hn_ref, wl_ref, bl_ref,
                  wr_ref, g_ref, bb_ref, out_ref):
    deg = jnp.clip(d0_ref[...][:, 0:1] + d1_ref[...][:, 0:1], 1.0, None)
    agg = (s0_ref[...] + s1_ref[...]) / deg
    hn = hn_ref[...]
    h = (jnp.dot(agg, wl_ref[...], preferred_element_type=jnp.float32)
         + bl_ref[...]
         + jnp.dot(hn, wr_ref[...], preferred_element_type=jnp.float32))
    out_ref[...] = _relu_ln(h, g_ref[...], bb_ref[...])


def _attn_body(s0_ref, s1_ref, d0_ref, d1_ref, hn_ref, wl_ref, bl_ref,
               wr_ref, g_ref, bb_ref,
               q_ref, dm_ref, sa_ref, wk_ref, bk_ref, wv_ref, bv_ref,
               w2_ref, b2_ref, wd_ref, bd_ref, out_ref):
    deg = jnp.clip(d0_ref[...][:, 0:1] + d1_ref[...][:, 0:1], 1.0, None)
    agg = (s0_ref[...] + s1_ref[...]) / deg
    hcomb = (jnp.dot(agg, wl_ref[...], preferred_element_type=jnp.float32)
             + bl_ref[...]
             + jnp.dot(hn_ref[...], wr_ref[...],
                       preferred_element_type=jnp.float32))
    hfin = _relu_ln(hcomb, g_ref[...], bb_ref[...])
    sa = sa_ref[...]
    cnt = sa[:, HID:HID + 1]
    avg = sa[:, :HID] / jnp.clip(cnt, 1.0, None)
    km = jnp.dot(avg, wk_ref[...], preferred_element_type=jnp.float32) + bk_ref[...]
    vm = jnp.dot(avg, wv_ref[...], preferred_element_type=jnp.float32) + bv_ref[...]
    lc = jnp.log(cnt)
    dmf = dm_ref[...].astype(jnp.float32) * wd_ref[0, 0] + bd_ref[0, 0]
    w2 = w2_ref[...]
    acc = (jnp.dot(hfin, w2[:HID, :], preferred_element_type=jnp.float32)
           + b2_ref[...])
    qs = q_ref[...] * SCALE
    ones_col = jnp.ones((RB, 1), jnp.float32)
    for h in range(HEADS):
        c0 = h * DH
        qh = jnp.concatenate([qs[:, c0:c0 + DH], ones_col], axis=1)
        kh = jnp.concatenate([km[:, c0:c0 + DH], lc], axis=1)
        dots = lax.dot_general(qh, kh, (((1,), (1,)), ((), ())),
                               preferred_element_type=jnp.float32) + dmf
        m = jnp.max(dots, axis=-1, keepdims=True)
        e = jnp.exp(dots - m)
        p = e / jnp.sum(e, axis=-1, keepdims=True)
        oh = jnp.dot(p, vm[:, c0:c0 + DH], preferred_element_type=jnp.float32)
        acc += jnp.dot(oh, w2[HID + c0:HID + c0 + DH, :],
                       preferred_element_type=jnp.float32)
    out_ref[...] = acc


def _row_spec():
    return pl.BlockSpec((RB, HID), lambda i: (i, 0))


def _const_spec(shape):
    return pl.BlockSpec(shape, lambda i: tuple(0 for _ in shape))


def _pre_call(x, w1, b1, g, b, wp, bp, wq, bq):
    return pl.pallas_call(
        _pre_body,
        grid=(GRID,),
        in_specs=[_row_spec()] + [_const_spec(a.shape)
                                  for a in (w1, b1, g, b, wp, bp, wq, bq)],
        out_specs=[_row_spec(), _row_spec()],
        out_shape=[jax.ShapeDtypeStruct((N, HID), jnp.float32)] * 2,
    )(x, w1, b1, g, b, wp, bp, wq, bq)


def _comm_call(ntc3, x):
    return pl.pallas_call(
        _comm_body,
        grid=(GRID,),
        in_specs=[pl.BlockSpec((1, 1, RB), lambda i: (i, 0, 0)), _row_spec()],
        out_specs=_const_spec((K, 2 * HID)),
        out_shape=jax.ShapeDtypeStruct((K, 2 * HID), jnp.float32),
    )(ntc3, x)


def _combine_call(s0, s1, d0, d1, hn, wl, bl, wr, g, b):
    return pl.pallas_call(
        _combine_body,
        grid=(GRID,),
        in_specs=[_row_spec(), _row_spec(),
                  pl.BlockSpec((RB, 16), lambda i: (i, 0)),
                  pl.BlockSpec((RB, 16), lambda i: (i, 0)),
                  _row_spec()] + [_const_spec(a.shape)
                                  for a in (wl, bl, wr, g, b)],
        out_specs=_row_spec(),
        out_shape=jax.ShapeDtypeStruct((N, HID), jnp.float32),
    )(s0, s1, d0, d1, hn, wl, bl, wr, g, b)


def _attn_call(s0, s1, d0, d1, hn, wl, bl, wr, g, b,
               q, dm, sa, wk, bk, wv, bv, w2, b2, wd, bd):
    return pl.pallas_call(
        _attn_body,
        grid=(GRID,),
        in_specs=[_row_spec(), _row_spec(),
                  pl.BlockSpec((RB, 16), lambda i: (i, 0)),
                  pl.BlockSpec((RB, 16), lambda i: (i, 0)),
                  _row_spec()]
                 + [_const_spec(a.shape) for a in (wl, bl, wr, g, b)]
                 + [_row_spec(), pl.BlockSpec((RB, K), lambda i: (i, 0))]
                 + [_const_spec(a.shape)
                    for a in (sa, wk, bk, wv, bv, w2, b2, wd, bd)],
        out_specs=_row_spec(),
        out_shape=jax.ShapeDtypeStruct((N, HID), jnp.float32),
    )(s0, s1, d0, d1, hn, wl, bl, wr, g, b,
      q, dm, sa, wk, bk, wv, bv, w2, b2, wd, bd)


def kernel(x, edge_index, distance_matrix, nodes_to_community_tensor, W1, b1,
           conv_gamma, conv_beta, Wl, bl, Wr, gamma, beta, Wp, bp, Wq, bq, Wk,
           bk, Wv, bv, w_dis, b_dis, W2, b2):
    r = lambda v: v.reshape(1, -1)
    pad = EPAD - E
    src_p = jnp.concatenate([edge_index[0], jnp.zeros((pad,), jnp.int32)])
    dst_p = jnp.concatenate([edge_index[1], jnp.full((pad,), N, jnp.int32)])
    src_p = src_p.reshape(EPAD // CH, CH)
    dst_p = dst_p.reshape(EPAD // CH, CH)
    zr = jnp.zeros((RPT, HID), jnp.float32)
    on = jnp.ones((CH, HID), jnp.float32)

    sc_deg, sc_edge = _get_sc_kernels()

    ntc3 = nodes_to_community_tensor.reshape(GRID, 1, RB)
    hn1, q = _pre_call(x, W1, r(b1), r(conv_gamma[0]), r(conv_beta[0]),
                       Wp, r(bp), Wq, r(bq))
    sa = _comm_call(ntc3, x)

    degs = sc_deg(dst_p, zr, on)
    d0 = degs[:N, :16]
    d1 = degs[NPAD:NPAD + N, :16]
    sums1 = sc_edge(hn1, src_p, dst_p, zr)
    hn2 = _combine_call(sums1[:N], sums1[NPAD:NPAD + N], d0, d1, hn1,
                        Wl[0], r(bl[0]), Wr[0], r(conv_gamma[1]),
                        r(conv_beta[1]))

    sums2 = sc_edge(hn2, src_p, dst_p, zr)

    out = _attn_call(sums2[:N], sums2[NPAD:NPAD + N], d0, d1, hn2,
                     Wl[1], r(bl[1]), Wr[1], r(gamma), r(beta),
                     q, distance_matrix, sa, Wk, r(bk), Wv, r(bv),
                     W2, r(b2), w_dis.reshape(1, 1), b_dis.reshape(1, 1))
    return out
